# Initial kernel scaffold; baseline (speedup 1.0000x reference)
import jax
import jax.numpy as jnp
from jax.experimental import pallas as pl


def kernel(input, W, b):
    raise NotImplementedError("write your pallas kernel here")


# speedup vs baseline: 2.9901x; 2.9901x over previous
import functools

import jax
import jax.numpy as jnp
from jax import lax
from jax.experimental import pallas as pl
from jax.experimental.pallas import tpu as pltpu
from jax.experimental.pallas import tpu_sc as plsc

M = 64
N = 64
B = 128
FLAT = M * N
NW = 32
SPW = B // NW


def _mm_body(x_ref, w_ref, b_ref, o_ref):
    o_ref[...] = jnp.dot(
        x_ref[...], w_ref[...],
        preferred_element_type=jnp.float32,
    ) + b_ref[...]


def _matmul(x, W, b2d):
    NB = 8
    BN = FLAT // NB
    return pl.pallas_call(
        _mm_body,
        grid=(NB,),
        in_specs=[
            pl.BlockSpec((B, FLAT), lambda n: (0, 0)),
            pl.BlockSpec((FLAT, BN), lambda n: (0, n)),
            pl.BlockSpec((1, BN), lambda n: (0, n)),
        ],
        out_specs=pl.BlockSpec((B, BN), lambda n: (0, n)),
        out_shape=jax.ShapeDtypeStruct((B, FLAT), jnp.float32),
    )(x, W, b2d)


def _ce(ak, av, bk, bv):
    m = ak <= bk
    return (jnp.where(m, ak, bk), jnp.where(m, av, bv),
            jnp.where(m, bk, ak), jnp.where(m, bv, av))


def _rev(x):
    return lax.rev(x, (0,))


def _sort64(ks, vs):
    s = [plsc.sort_key_val(ks[c], vs[c]) for c in range(4)]

    def merge16(a, b):
        lok, lov, hik, hiv = _ce(a[0], a[1], _rev(b[0]), _rev(b[1]))
        return plsc.sort_key_val(lok, lov), plsc.sort_key_val(hik, hiv)

    a0, a1 = merge16(s[0], s[1])
    b0, b1 = merge16(s[2], s[3])
    l0k, l0v, h0k, h0v = _ce(a0[0], a0[1], _rev(b1[0]), _rev(b1[1]))
    l1k, l1v, h1k, h1v = _ce(a1[0], a1[1], _rev(b0[0]), _rev(b0[1]))
    llk, llv, lhk, lhv = _ce(l0k, l0v, l1k, l1v)
    hlk, hlv, hhk, hhv = _ce(h0k, h0v, h1k, h1v)
    outs = [plsc.sort_key_val(llk, llv), plsc.sort_key_val(lhk, lhv),
            plsc.sort_key_val(hlk, hlv), plsc.sort_key_val(hhk, hhv)]
    return [o[1] for o in outs]


_SC_SCRATCH = [
    pltpu.VMEM((FLAT,), jnp.float32),
    pltpu.VMEM((FLAT,), jnp.float32),
    pltpu.VMEM((FLAT,), jnp.int32),
    pltpu.VMEM((FLAT,), jnp.int32),
    pltpu.VMEM((FLAT,), jnp.float32),
]


def _sc_body(o_hbm, in_hbm, out_hbm, o_v, in_v, sx_v, sy_v, res_v):
    wid = lax.axis_index("s") * 2 + lax.axis_index("c")
    iota = lax.iota(jnp.int32, 16)

    def sample_body(i, carry):
        base = (wid * SPW + i) * FLAT
        pltpu.sync_copy(o_hbm.at[pl.ds(base, FLAT)], o_v)
        pltpu.sync_copy(in_hbm.at[pl.ds(base, FLAT)], in_v)

        def row_body(r, c2):
            rb = r * 64
            ks = [o_v[pl.ds(rb + 16 * c, 16)] for c in range(4)]
            vs = [iota + 16 * c for c in range(4)]
            perm = _sort64(ks, vs)
            for c in range(4):
                sx_v[pl.ds(rb + 16 * c, 16)] = perm[c]
            return c2
        lax.fori_loop(0, 64, row_body, 0)

        def col_body(j, c2):
            ks = [plsc.load_gather(o_v, [(iota + 16 * c) * 64 + j])
                  for c in range(4)]
            vs = [iota + 16 * c for c in range(4)]
            perm = _sort64(ks, vs)
            for c in range(4):
                plsc.store_scatter(sy_v, [(iota + 16 * c) * 64 + j], perm[c])
            return c2
        lax.fori_loop(0, 64, col_body, 0)

        def out_body(t, c2):
            r = sy_v[pl.ds(t * 16, 16)]
            j = ((t & 3) << 4) + iota
            cc = plsc.load_gather(sx_v, [r * 64 + j])
            val = plsc.load_gather(in_v, [r * 64 + cc])
            res_v[pl.ds(t * 16, 16)] = val
            return c2
        lax.fori_loop(0, 256, out_body, 0)

        pltpu.sync_copy(res_v, out_hbm.at[pl.ds(base, FLAT)])
        return carry

    lax.fori_loop(0, SPW, sample_body, 0)


_sc_permute = functools.partial(
    pl.kernel,
    out_type=jax.ShapeDtypeStruct((B * FLAT,), jnp.float32),
    mesh=plsc.VectorSubcoreMesh(
        core_axis_name="c", subcore_axis_name="s", num_cores=2, num_subcores=16),
    compiler_params=pltpu.CompilerParams(needs_layout_passes=False),
    scratch_types=_SC_SCRATCH,
)(_sc_body)


def kernel(input, W, b):
    x = jnp.reshape(input.astype(jnp.float32), (B, FLAT))
    o = _matmul(x, W, jnp.reshape(b, (1, FLAT)))
    res = _sc_permute(jnp.reshape(o, (B * FLAT,)), jnp.reshape(x, (B * FLAT,)))
    return jnp.reshape(res, (B, M, N))
